# baseline (speedup 1.0000x reference)
import functools

import jax
import jax.numpy as jnp
from jax import lax
from jax.experimental import pallas as pl
from jax.experimental.pallas import tpu as pltpu
from jax.experimental.pallas import tpu_sc as plsc

N, E, D, H, C = 10000, 320000, 128, 128, 10
NC, NS = 2, 16
NW = NC * NS
EPW = E // NW
CHUNK = 125
CHUNKS = EPW // CHUNK
NPAD = 10112
ROWS = NPAD // NS
W1 = 160
BN = 1000


def _make_segsum(width):
  mesh = plsc.VectorSubcoreMesh(core_axis_name="c", subcore_axis_name="s")

  @functools.partial(
      pl.kernel, mesh=mesh,
      out_type=jax.ShapeDtypeStruct((NC, NPAD, width), jnp.bfloat16),
      scratch_types=[
          pltpu.VMEM((CHUNKS, CHUNK), jnp.int32),
          pltpu.VMEM((CHUNKS, CHUNK), jnp.int32),
          pltpu.VMEM((CHUNK, width), jnp.bfloat16),
          pltpu.VMEM((CHUNK, width), jnp.bfloat16),
          pltpu.VMEM_SHARED((NPAD, width), jnp.bfloat16),
          pltpu.SemaphoreType.DMA,
          pltpu.SemaphoreType.DMA,
      ],
      compiler_params=pltpu.CompilerParams(
          use_tc_tiling_on_sc=False, needs_layout_passes=False))
  def seg(msg_hbm, ei_hbm, zero_hbm, out_hbm,
          src_v, dst_v, bufa, bufb, acc, sema, semb):
    c = lax.axis_index("c")
    s = lax.axis_index("s")
    rbase = s * ROWS
    pltpu.sync_copy(ei_hbm.at[0, c, s], src_v)
    pltpu.sync_copy(ei_hbm.at[1, c, s], dst_v)
    pltpu.sync_copy(zero_hbm, acc.at[pl.ds(rbase, ROWS)])

    plsc.subcore_barrier()
    pltpu.async_copy(msg_hbm.at[src_v.at[0]], bufa, sema)

    def body(g, carry):
      j0 = 2 * g
      j1 = j0 + 1
      cpb = pltpu.async_copy(msg_hbm.at[src_v.at[j1]], bufb, semb)
      pltpu.make_async_copy(msg_hbm.at[src_v.at[j0]], bufa, sema).wait()
      pltpu.sync_copy(bufa, acc.at[dst_v.at[j0]], add=True)

      @pl.when(j0 + 2 < CHUNKS)
      def _prefetch():
        pltpu.async_copy(msg_hbm.at[src_v.at[j0 + 2]], bufa, sema)

      cpb.wait()
      pltpu.sync_copy(bufb, acc.at[dst_v.at[j1]], add=True)
      return carry

    lax.fori_loop(0, CHUNKS // 2, body, 0)

    plsc.subcore_barrier()
    pltpu.sync_copy(acc.at[pl.ds(rbase, ROWS)],
                    out_hbm.at[c].at[pl.ds(rbase, ROWS)])

  return seg


_segsum_w1 = _make_segsum(W1)
_segsum_h = _make_segsum(H)


def _tc_pre_body(x_ref, wn_ref, m_ref):
  mm = jnp.dot(x_ref[...], wn_ref[...], preferred_element_type=jnp.float32)
  col = lax.broadcasted_iota(jnp.int32, (BN, W1 - H), 1)
  tail = jnp.where(col == 0, 1.0, 0.0)
  m_ref[...] = jnp.concatenate([mm, tail], axis=1).astype(jnp.bfloat16)


def _tc_mid_body(p_ref, x_ref, w1s_ref, b1_ref, wn_ref, ws_ref, b2_ref,
                 m_ref, s_ref, dinv_ref):
  p = p_ref[...].astype(jnp.float32)
  a = p[0] + p[1]
  deg = a[:, H:H + 1]
  dinv = 1.0 / jnp.maximum(deg, 1.0)
  s1 = jnp.dot(x_ref[...], w1s_ref[...],
               preferred_element_type=jnp.float32) + b1_ref[...]
  h1 = jnp.maximum(s1 + a[:, :H] * dinv, 0.0)
  m_ref[...] = jnp.dot(h1, wn_ref[...],
                       preferred_element_type=jnp.float32).astype(jnp.bfloat16)
  s_ref[...] = jnp.dot(h1, ws_ref[...], preferred_element_type=jnp.float32) + b2_ref[...]
  dinv_ref[...] = jnp.broadcast_to(dinv, (BN, 8))


def _tc_out_body(p_ref, s2_ref, dinv_ref, wo_ref, bo_ref, out_ref):
  p = p_ref[...].astype(jnp.float32)
  a = p[0] + p[1]
  h2 = jnp.maximum(s2_ref[...] + a * dinv_ref[...][:, 0:1], 0.0)
  logits = jnp.dot(h2, wo_ref[...], preferred_element_type=jnp.float32) + bo_ref[...]
  out_ref[...] = jnp.clip(logits, -4.0, 4.0)


_GRID = (N // BN,)
_FULL = lambda i: (0, 0)
_ROWB = lambda i: (i, 0)

_tc_pre = pl.pallas_call(
    _tc_pre_body,
    grid=_GRID,
    in_specs=[
        pl.BlockSpec((BN, D), _ROWB),
        pl.BlockSpec((D, H), _FULL),
    ],
    out_specs=pl.BlockSpec((BN, W1), _ROWB),
    out_shape=jax.ShapeDtypeStruct((N, W1), jnp.bfloat16),
)

_tc_mid = pl.pallas_call(
    _tc_mid_body,
    grid=_GRID,
    in_specs=[
        pl.BlockSpec((NC, BN, W1), lambda i: (0, i, 0)),
        pl.BlockSpec((BN, D), _ROWB),
        pl.BlockSpec((D, H), _FULL),
        pl.BlockSpec((1, H), _FULL),
        pl.BlockSpec((H, H), _FULL),
        pl.BlockSpec((H, H), _FULL),
        pl.BlockSpec((1, H), _FULL),
    ],
    out_specs=[pl.BlockSpec((BN, H), _ROWB), pl.BlockSpec((BN, H), _ROWB),
               pl.BlockSpec((BN, 8), _ROWB)],
    out_shape=[jax.ShapeDtypeStruct((N, H), jnp.bfloat16),
               jax.ShapeDtypeStruct((N, H), jnp.float32),
               jax.ShapeDtypeStruct((N, 8), jnp.float32)],
)

_tc_out = pl.pallas_call(
    _tc_out_body,
    grid=_GRID,
    in_specs=[
        pl.BlockSpec((NC, BN, H), lambda i: (0, i, 0)),
        pl.BlockSpec((BN, H), _ROWB),
        pl.BlockSpec((BN, 8), _ROWB),
        pl.BlockSpec((H, C), _FULL),
        pl.BlockSpec((1, C), _FULL),
    ],
    out_specs=pl.BlockSpec((BN, C), _ROWB),
    out_shape=jax.ShapeDtypeStruct((N, C), jnp.float32),
)


def kernel(x, edge_index, y, W1_self, W1_neigh, b1, W2_self, W2_neigh, b2,
           W_out, b_out):
  ei = edge_index.reshape(2, NC, NS, CHUNKS, CHUNK)
  zeros_w1 = jnp.zeros((ROWS, W1), jnp.bfloat16)
  zeros_h = jnp.zeros((ROWS, H), jnp.bfloat16)

  m1 = _tc_pre(x, W1_neigh)
  parts1 = _segsum_w1(m1, ei, zeros_w1)
  m2, s2, dinv = _tc_mid(parts1, x, W1_self, b1.reshape(1, H),
                         W2_neigh, W2_self, b2.reshape(1, H))
  parts2 = _segsum_h(m2, ei, zeros_h)
  logits = _tc_out(parts2, s2, dinv, W_out, b_out.reshape(1, C))
  return (logits, y)


# speedup vs baseline: 1.1064x; 1.1064x over previous
import functools

import jax
import jax.numpy as jnp
from jax import lax
from jax.experimental import pallas as pl
from jax.experimental.pallas import tpu as pltpu
from jax.experimental.pallas import tpu_sc as plsc

N, E, D, H, C = 10000, 320000, 128, 128, 10
NC, NS = 2, 16
NW = NC * NS
EPW = E // NW
CHUNK = 125
CHUNKS = EPW // CHUNK
NPAD = 10112
ROWS = NPAD // NS
DW = 16
BN = 1000


def _make_segsum(with_deg):
  mesh = plsc.VectorSubcoreMesh(core_axis_name="c", subcore_axis_name="s")
  out_type = [jax.ShapeDtypeStruct((NC, NPAD, H), jnp.bfloat16)]
  scratch = [
      pltpu.VMEM((CHUNKS, CHUNK), jnp.int32),
      pltpu.VMEM((CHUNKS, CHUNK), jnp.int32),
      pltpu.VMEM((CHUNK, H), jnp.bfloat16),
      pltpu.VMEM((CHUNK, H), jnp.bfloat16),
      pltpu.VMEM_SHARED((NPAD, H), jnp.bfloat16),
      pltpu.SemaphoreType.DMA,
      pltpu.SemaphoreType.DMA,
      pltpu.SemaphoreType.DMA,
  ]
  if with_deg:
    out_type.append(jax.ShapeDtypeStruct((NC, NPAD, DW), jnp.float32))
    scratch.insert(4, pltpu.VMEM((CHUNK, DW), jnp.float32))
    scratch.insert(5, pltpu.VMEM_SHARED((NPAD, DW), jnp.float32))

  @functools.partial(pl.kernel, mesh=mesh, out_type=out_type,
                     scratch_types=scratch,
                     compiler_params=pltpu.CompilerParams(
                         use_tc_tiling_on_sc=False,
                         needs_layout_passes=False))
  def seg(*refs):
    if with_deg:
      (msg_hbm, ei_hbm, zero_hbm, dzero_hbm, out_hbm, deg_hbm,
       src_v, dst_v, bufa, bufb, ones_v, dacc, acc, sema, semb, semd) = refs
    else:
      (msg_hbm, ei_hbm, zero_hbm, out_hbm,
       src_v, dst_v, bufa, bufb, acc, sema, semb, semd) = refs
    c = lax.axis_index("c")
    s = lax.axis_index("s")
    rbase = s * ROWS
    pltpu.sync_copy(ei_hbm.at[0, c, s], src_v)
    pltpu.sync_copy(ei_hbm.at[1, c, s], dst_v)
    pltpu.sync_copy(zero_hbm, acc.at[pl.ds(rbase, ROWS)])
    if with_deg:
      pltpu.sync_copy(dzero_hbm, dacc.at[pl.ds(rbase, ROWS)])
      ones16 = jnp.ones((16,), jnp.float32)
      def obody(j, carry):
        ones_v[j, pl.ds(0, DW)] = ones16
        return carry
      lax.fori_loop(0, CHUNK, obody, 0)

    plsc.subcore_barrier()
    pltpu.async_copy(msg_hbm.at[src_v.at[0]], bufa, sema)

    def _scatter(buf, j):
      if with_deg:
        dcp = pltpu.async_copy(ones_v, dacc.at[dst_v.at[j]], semd, add=True)
        pltpu.sync_copy(buf, acc.at[dst_v.at[j]], add=True)
        dcp.wait()
      else:
        pltpu.sync_copy(buf, acc.at[dst_v.at[j]], add=True)

    def body(g, carry):
      j0 = 2 * g
      j1 = j0 + 1
      cpb = pltpu.async_copy(msg_hbm.at[src_v.at[j1]], bufb, semb)
      pltpu.make_async_copy(msg_hbm.at[src_v.at[j0]], bufa, sema).wait()
      _scatter(bufa, j0)

      @pl.when(j0 + 2 < CHUNKS)
      def _prefetch():
        pltpu.async_copy(msg_hbm.at[src_v.at[j0 + 2]], bufa, sema)

      cpb.wait()
      _scatter(bufb, j1)
      return carry

    lax.fori_loop(0, CHUNKS // 2, body, 0)

    plsc.subcore_barrier()
    pltpu.sync_copy(acc.at[pl.ds(rbase, ROWS)],
                    out_hbm.at[c].at[pl.ds(rbase, ROWS)])
    if with_deg:
      pltpu.sync_copy(dacc.at[pl.ds(rbase, ROWS)],
                      deg_hbm.at[c].at[pl.ds(rbase, ROWS)])

  return seg


_segsum_deg = _make_segsum(True)
_segsum_h = _make_segsum(False)


def _tc_pre_body(x_ref, wn_ref, m_ref):
  mm = jnp.dot(x_ref[...], wn_ref[...], preferred_element_type=jnp.float32)
  m_ref[...] = mm.astype(jnp.bfloat16)


def _tc_mid_body(p_ref, dp_ref, x_ref, w1s_ref, b1_ref, wn_ref, ws_ref, b2_ref,
                 m_ref, s_ref, dinv_ref):
  p = p_ref[...].astype(jnp.float32)
  a = p[0] + p[1]
  dp = dp_ref[...]
  deg = (dp[0] + dp[1])[:, 0:1]
  dinv = 1.0 / jnp.maximum(deg, 1.0)
  s1 = jnp.dot(x_ref[...], w1s_ref[...],
               preferred_element_type=jnp.float32) + b1_ref[...]
  h1 = jnp.maximum(s1 + a * dinv, 0.0)
  m_ref[...] = jnp.dot(h1, wn_ref[...],
                       preferred_element_type=jnp.float32).astype(jnp.bfloat16)
  s_ref[...] = jnp.dot(h1, ws_ref[...], preferred_element_type=jnp.float32) + b2_ref[...]
  dinv_ref[...] = jnp.broadcast_to(dinv, (BN, 8))


def _tc_out_body(p_ref, s2_ref, dinv_ref, wo_ref, bo_ref, out_ref):
  p = p_ref[...].astype(jnp.float32)
  a = p[0] + p[1]
  h2 = jnp.maximum(s2_ref[...] + a * dinv_ref[...][:, 0:1], 0.0)
  logits = jnp.dot(h2, wo_ref[...], preferred_element_type=jnp.float32) + bo_ref[...]
  out_ref[...] = jnp.clip(logits, -4.0, 4.0)


_GRID = (N // BN,)
_FULL = lambda i: (0, 0)
_ROWB = lambda i: (i, 0)

_tc_pre = pl.pallas_call(
    _tc_pre_body,
    grid=_GRID,
    in_specs=[
        pl.BlockSpec((BN, D), _ROWB),
        pl.BlockSpec((D, H), _FULL),
    ],
    out_specs=pl.BlockSpec((BN, H), _ROWB),
    out_shape=jax.ShapeDtypeStruct((N, H), jnp.bfloat16),
)

_tc_mid = pl.pallas_call(
    _tc_mid_body,
    grid=_GRID,
    in_specs=[
        pl.BlockSpec((NC, BN, H), lambda i: (0, i, 0)),
        pl.BlockSpec((NC, BN, DW), lambda i: (0, i, 0)),
        pl.BlockSpec((BN, D), _ROWB),
        pl.BlockSpec((D, H), _FULL),
        pl.BlockSpec((1, H), _FULL),
        pl.BlockSpec((H, H), _FULL),
        pl.BlockSpec((H, H), _FULL),
        pl.BlockSpec((1, H), _FULL),
    ],
    out_specs=[pl.BlockSpec((BN, H), _ROWB), pl.BlockSpec((BN, H), _ROWB),
               pl.BlockSpec((BN, 8), _ROWB)],
    out_shape=[jax.ShapeDtypeStruct((N, H), jnp.bfloat16),
               jax.ShapeDtypeStruct((N, H), jnp.float32),
               jax.ShapeDtypeStruct((N, 8), jnp.float32)],
)

_tc_out = pl.pallas_call(
    _tc_out_body,
    grid=_GRID,
    in_specs=[
        pl.BlockSpec((NC, BN, H), lambda i: (0, i, 0)),
        pl.BlockSpec((BN, H), _ROWB),
        pl.BlockSpec((BN, 8), _ROWB),
        pl.BlockSpec((H, C), _FULL),
        pl.BlockSpec((1, C), _FULL),
    ],
    out_specs=pl.BlockSpec((BN, C), _ROWB),
    out_shape=jax.ShapeDtypeStruct((N, C), jnp.float32),
)


def kernel(x, edge_index, y, W1_self, W1_neigh, b1, W2_self, W2_neigh, b2,
           W_out, b_out):
  ei = edge_index.reshape(2, NC, NS, CHUNKS, CHUNK)
  zeros_h = jnp.zeros((ROWS, H), jnp.bfloat16)
  dzeros = jnp.zeros((ROWS, DW), jnp.float32)

  m1 = _tc_pre(x, W1_neigh)
  parts1, degp = _segsum_deg(m1, ei, zeros_h, dzeros)
  m2, s2, dinv = _tc_mid(parts1, degp, x, W1_self, b1.reshape(1, H),
                         W2_neigh, W2_self, b2.reshape(1, H))
  (parts2,) = _segsum_h(m2, ei, zeros_h)
  logits = _tc_out(parts2, s2, dinv, W_out, b_out.reshape(1, C))
  return (logits, y)


# speedup vs baseline: 1.1348x; 1.0257x over previous
import functools

import jax
import jax.numpy as jnp
from jax import lax
from jax.experimental import pallas as pl
from jax.experimental.pallas import tpu as pltpu
from jax.experimental.pallas import tpu_sc as plsc

N, E, D, H, C = 10000, 320000, 128, 128, 10
NC, NS = 2, 16
NW = NC * NS
EPW = E // NW
CHUNK = 125
CHUNKS = EPW // CHUNK
NPAD = 10112
ROWS = NPAD // NS
DW = 16
BN = 1000


def _make_segsum(with_deg):
  mesh = plsc.VectorSubcoreMesh(core_axis_name="c", subcore_axis_name="s")
  out_type = [jax.ShapeDtypeStruct((NC, NPAD, H), jnp.bfloat16)]
  scratch = [
      pltpu.VMEM((CHUNKS, CHUNK), jnp.int32),
      pltpu.VMEM((CHUNKS, CHUNK), jnp.int32),
      pltpu.VMEM((CHUNK, H), jnp.bfloat16),
      pltpu.VMEM((CHUNK, H), jnp.bfloat16),
      pltpu.VMEM((CHUNK, H), jnp.bfloat16),
      pltpu.VMEM((CHUNK, H), jnp.bfloat16),
      pltpu.VMEM_SHARED((NPAD, H), jnp.bfloat16),
  ] + [pltpu.SemaphoreType.DMA] * 8
  if with_deg:
    out_type.append(jax.ShapeDtypeStruct((NC, NPAD, DW), jnp.float32))
    scratch.insert(6, pltpu.VMEM((CHUNK, DW), jnp.float32))
    scratch.insert(7, pltpu.VMEM_SHARED((NPAD, DW), jnp.float32))

  @functools.partial(pl.kernel, mesh=mesh, out_type=out_type,
                     scratch_types=scratch,
                     compiler_params=pltpu.CompilerParams(
                         use_tc_tiling_on_sc=False,
                         needs_layout_passes=False))
  def seg(*refs):
    if with_deg:
      (msg_hbm, ei_hbm, zero_hbm, dzero_hbm, out_hbm, deg_hbm,
       src_v, dst_v, b0, b1, b2, b3, ones_v, dacc, acc, *sems) = refs
    else:
      (msg_hbm, ei_hbm, zero_hbm, out_hbm,
       src_v, dst_v, b0, b1, b2, b3, acc, *sems) = refs
    bufs = [b0, b1, b2, b3]
    gsems, ssems = sems[:4], sems[4:]
    c = lax.axis_index("c")
    s = lax.axis_index("s")
    rbase = s * ROWS
    pltpu.sync_copy(ei_hbm.at[0, c, s], src_v)
    pltpu.sync_copy(ei_hbm.at[1, c, s], dst_v)
    pltpu.sync_copy(zero_hbm, acc.at[pl.ds(rbase, ROWS)])
    if with_deg:
      pltpu.sync_copy(dzero_hbm, dacc.at[pl.ds(rbase, ROWS)])
      ones16 = jnp.ones((16,), jnp.float32)
      def obody(j, carry):
        ones_v[j, pl.ds(0, DW)] = ones16
        return carry
      lax.fori_loop(0, CHUNK, obody, 0)

    plsc.subcore_barrier()
    pltpu.async_copy(msg_hbm.at[src_v.at[0]], b0, gsems[0])
    pltpu.async_copy(msg_hbm.at[src_v.at[1]], b1, gsems[1])

    def _wait_scatter(buf, j, ss):
      pltpu.make_async_copy(buf, acc.at[dst_v.at[j]], ss).wait()
      if with_deg:
        pltpu.make_async_copy(ones_v, dacc.at[dst_v.at[j]], ss).wait()

    def body(g, carry):
      for k in range(4):
        j = 4 * g + k
        buf, ss = bufs[k], ssems[k]
        pltpu.make_async_copy(msg_hbm.at[src_v.at[j]], buf, gsems[k]).wait()
        pltpu.async_copy(buf, acc.at[dst_v.at[j]], ss, add=True)
        if with_deg:
          pltpu.async_copy(ones_v, dacc.at[dst_v.at[j]], ss, add=True)
        jn = j + 2
        kn = (k + 2) % 4

        @pl.when(jn < CHUNKS)
        def _prefetch():
          @pl.when(j >= 2)
          def _free():
            _wait_scatter(bufs[kn], j - 2, ssems[kn])
          pltpu.async_copy(msg_hbm.at[src_v.at[jn]], bufs[kn], gsems[kn])
      return carry

    lax.fori_loop(0, CHUNKS // 4, body, 0)
    for k in range(4):
      _wait_scatter(bufs[k], CHUNKS - 4 + k, ssems[k])

    plsc.subcore_barrier()
    pltpu.sync_copy(acc.at[pl.ds(rbase, ROWS)],
                    out_hbm.at[c].at[pl.ds(rbase, ROWS)])
    if with_deg:
      pltpu.sync_copy(dacc.at[pl.ds(rbase, ROWS)],
                      deg_hbm.at[c].at[pl.ds(rbase, ROWS)])

  return seg


_segsum_deg = _make_segsum(True)
_segsum_h = _make_segsum(False)


def _tc_pre_body(x_ref, wn_ref, m_ref):
  mm = jnp.dot(x_ref[...], wn_ref[...], preferred_element_type=jnp.float32)
  m_ref[...] = mm.astype(jnp.bfloat16)


def _tc_mid_body(p_ref, dp_ref, x_ref, w1s_ref, b1_ref, wn_ref, ws_ref, b2_ref,
                 m_ref, s_ref, dinv_ref):
  p = p_ref[...].astype(jnp.float32)
  a = p[0] + p[1]
  dp = dp_ref[...]
  deg = (dp[0] + dp[1])[:, 0:1]
  dinv = 1.0 / jnp.maximum(deg, 1.0)
  s1 = jnp.dot(x_ref[...], w1s_ref[...],
               preferred_element_type=jnp.float32) + b1_ref[...]
  h1 = jnp.maximum(s1 + a * dinv, 0.0)
  m_ref[...] = jnp.dot(h1, wn_ref[...],
                       preferred_element_type=jnp.float32).astype(jnp.bfloat16)
  s_ref[...] = jnp.dot(h1, ws_ref[...], preferred_element_type=jnp.float32) + b2_ref[...]
  dinv_ref[...] = jnp.broadcast_to(dinv, (BN, 8))


def _tc_out_body(p_ref, s2_ref, dinv_ref, wo_ref, bo_ref, out_ref):
  p = p_ref[...].astype(jnp.float32)
  a = p[0] + p[1]
  h2 = jnp.maximum(s2_ref[...] + a * dinv_ref[...][:, 0:1], 0.0)
  logits = jnp.dot(h2, wo_ref[...], preferred_element_type=jnp.float32) + bo_ref[...]
  out_ref[...] = jnp.clip(logits, -4.0, 4.0)


_GRID = (N // BN,)
_FULL = lambda i: (0, 0)
_ROWB = lambda i: (i, 0)

_tc_pre = pl.pallas_call(
    _tc_pre_body,
    grid=_GRID,
    in_specs=[
        pl.BlockSpec((BN, D), _ROWB),
        pl.BlockSpec((D, H), _FULL),
    ],
    out_specs=pl.BlockSpec((BN, H), _ROWB),
    out_shape=jax.ShapeDtypeStruct((N, H), jnp.bfloat16),
)

_tc_mid = pl.pallas_call(
    _tc_mid_body,
    grid=_GRID,
    in_specs=[
        pl.BlockSpec((NC, BN, H), lambda i: (0, i, 0)),
        pl.BlockSpec((NC, BN, DW), lambda i: (0, i, 0)),
        pl.BlockSpec((BN, D), _ROWB),
        pl.BlockSpec((D, H), _FULL),
        pl.BlockSpec((1, H), _FULL),
        pl.BlockSpec((H, H), _FULL),
        pl.BlockSpec((H, H), _FULL),
        pl.BlockSpec((1, H), _FULL),
    ],
    out_specs=[pl.BlockSpec((BN, H), _ROWB), pl.BlockSpec((BN, H), _ROWB),
               pl.BlockSpec((BN, 8), _ROWB)],
    out_shape=[jax.ShapeDtypeStruct((N, H), jnp.bfloat16),
               jax.ShapeDtypeStruct((N, H), jnp.float32),
               jax.ShapeDtypeStruct((N, 8), jnp.float32)],
)

_tc_out = pl.pallas_call(
    _tc_out_body,
    grid=_GRID,
    in_specs=[
        pl.BlockSpec((NC, BN, H), lambda i: (0, i, 0)),
        pl.BlockSpec((BN, H), _ROWB),
        pl.BlockSpec((BN, 8), _ROWB),
        pl.BlockSpec((H, C), _FULL),
        pl.BlockSpec((1, C), _FULL),
    ],
    out_specs=pl.BlockSpec((BN, C), _ROWB),
    out_shape=jax.ShapeDtypeStruct((N, C), jnp.float32),
)


def kernel(x, edge_index, y, W1_self, W1_neigh, b1, W2_self, W2_neigh, b2,
           W_out, b_out):
  ei = edge_index.reshape(2, NC, NS, CHUNKS, CHUNK)
  zeros_h = jnp.zeros((ROWS, H), jnp.bfloat16)
  dzeros = jnp.zeros((ROWS, DW), jnp.float32)

  m1 = _tc_pre(x, W1_neigh)
  parts1, degp = _segsum_deg(m1, ei, zeros_h, dzeros)
  m2, s2, dinv = _tc_mid(parts1, degp, x, W1_self, b1.reshape(1, H),
                         W2_neigh, W2_self, b2.reshape(1, H))
  (parts2,) = _segsum_h(m2, ei, zeros_h)
  logits = _tc_out(parts2, s2, dinv, W_out, b_out.reshape(1, C))
  return (logits, y)


# speedup vs baseline: 1.1620x; 1.0239x over previous
import functools

import jax
import jax.numpy as jnp
from jax import lax
from jax.experimental import pallas as pl
from jax.experimental.pallas import tpu as pltpu
from jax.experimental.pallas import tpu_sc as plsc

N, E, D, H, C = 10000, 320000, 128, 128, 10
NC, NS = 2, 16
NW = NC * NS
EPW = E // NW
CHUNK = 125
CHUNKS = EPW // CHUNK
NPAD = 10112
ROWS = NPAD // NS
DW = 16
BN = 2000


def _make_segsum(with_deg):
  mesh = plsc.VectorSubcoreMesh(core_axis_name="c", subcore_axis_name="s")
  out_type = [jax.ShapeDtypeStruct((NC, NPAD, H), jnp.bfloat16)]
  scratch = [
      pltpu.VMEM((CHUNKS, CHUNK), jnp.int32),
      pltpu.VMEM((CHUNKS, CHUNK), jnp.int32),
      pltpu.VMEM((CHUNK, H), jnp.bfloat16),
      pltpu.VMEM((CHUNK, H), jnp.bfloat16),
      pltpu.VMEM((CHUNK, H), jnp.bfloat16),
      pltpu.VMEM((CHUNK, H), jnp.bfloat16),
      pltpu.VMEM_SHARED((NPAD, H), jnp.bfloat16),
  ] + [pltpu.SemaphoreType.DMA] * 8
  if with_deg:
    out_type.append(jax.ShapeDtypeStruct((NC, NPAD, DW), jnp.float32))
    scratch.insert(6, pltpu.VMEM((CHUNK, DW), jnp.float32))
    scratch.insert(7, pltpu.VMEM_SHARED((NPAD, DW), jnp.float32))

  @functools.partial(pl.kernel, mesh=mesh, out_type=out_type,
                     scratch_types=scratch,
                     compiler_params=pltpu.CompilerParams(
                         use_tc_tiling_on_sc=False,
                         needs_layout_passes=False))
  def seg(*refs):
    if with_deg:
      (msg_hbm, ei_hbm, zero_hbm, dzero_hbm, out_hbm, deg_hbm,
       src_v, dst_v, b0, b1, b2, b3, ones_v, dacc, acc, *sems) = refs
    else:
      (msg_hbm, ei_hbm, zero_hbm, out_hbm,
       src_v, dst_v, b0, b1, b2, b3, acc, *sems) = refs
    bufs = [b0, b1, b2, b3]
    gsems, ssems = sems[:4], sems[4:]
    c = lax.axis_index("c")
    s = lax.axis_index("s")
    rbase = s * ROWS
    pltpu.sync_copy(ei_hbm.at[0, c, s], src_v)
    pltpu.sync_copy(ei_hbm.at[1, c, s], dst_v)
    pltpu.sync_copy(zero_hbm, acc.at[pl.ds(rbase, ROWS)])
    if with_deg:
      pltpu.sync_copy(dzero_hbm, dacc.at[pl.ds(rbase, ROWS)])
      ones16 = jnp.ones((16,), jnp.float32)
      def obody(j, carry):
        ones_v[j, pl.ds(0, DW)] = ones16
        return carry
      lax.fori_loop(0, CHUNK, obody, 0)

    plsc.subcore_barrier()
    pltpu.async_copy(msg_hbm.at[src_v.at[0]], b0, gsems[0])
    pltpu.async_copy(msg_hbm.at[src_v.at[1]], b1, gsems[1])

    def _wait_scatter(buf, j, ss):
      pltpu.make_async_copy(buf, acc.at[dst_v.at[j]], ss).wait()
      if with_deg:
        pltpu.make_async_copy(ones_v, dacc.at[dst_v.at[j]], ss).wait()

    def body(g, carry):
      for k in range(4):
        j = 4 * g + k
        buf, ss = bufs[k], ssems[k]
        pltpu.make_async_copy(msg_hbm.at[src_v.at[j]], buf, gsems[k]).wait()
        pltpu.async_copy(buf, acc.at[dst_v.at[j]], ss, add=True)
        if with_deg:
          pltpu.async_copy(ones_v, dacc.at[dst_v.at[j]], ss, add=True)
        jn = j + 2
        kn = (k + 2) % 4

        @pl.when(jn < CHUNKS)
        def _prefetch():
          @pl.when(j >= 2)
          def _free():
            _wait_scatter(bufs[kn], j - 2, ssems[kn])
          pltpu.async_copy(msg_hbm.at[src_v.at[jn]], bufs[kn], gsems[kn])
      return carry

    lax.fori_loop(0, CHUNKS // 4, body, 0)
    for k in range(4):
      _wait_scatter(bufs[k], CHUNKS - 4 + k, ssems[k])

    plsc.subcore_barrier()
    pltpu.sync_copy(acc.at[pl.ds(rbase, ROWS)],
                    out_hbm.at[c].at[pl.ds(rbase, ROWS)])
    if with_deg:
      pltpu.sync_copy(dacc.at[pl.ds(rbase, ROWS)],
                      deg_hbm.at[c].at[pl.ds(rbase, ROWS)])

  return seg


_segsum_deg = _make_segsum(True)
_segsum_h = _make_segsum(False)


def _tc_pre_body(x_ref, wn_ref, m_ref):
  mm = jnp.dot(x_ref[...], wn_ref[...], preferred_element_type=jnp.float32)
  m_ref[...] = mm.astype(jnp.bfloat16)


def _tc_mid_body(p_ref, dp_ref, x_ref, w1s_ref, b1_ref, wn_ref, ws_ref, b2_ref,
                 m_ref, s_ref, dinv_ref):
  p = p_ref[...].astype(jnp.float32)
  a = p[0] + p[1]
  dp = dp_ref[...]
  deg = (dp[0] + dp[1])[:, 0:1]
  dinv = 1.0 / jnp.maximum(deg, 1.0)
  s1 = jnp.dot(x_ref[...], w1s_ref[...],
               preferred_element_type=jnp.float32) + b1_ref[...]
  h1 = jnp.maximum(s1 + a * dinv, 0.0)
  m_ref[...] = jnp.dot(h1, wn_ref[...],
                       preferred_element_type=jnp.float32).astype(jnp.bfloat16)
  s_ref[...] = jnp.dot(h1, ws_ref[...], preferred_element_type=jnp.float32) + b2_ref[...]
  dinv_ref[...] = jnp.broadcast_to(dinv, (BN, 8))


def _tc_out_body(p_ref, s2_ref, dinv_ref, wo_ref, bo_ref, out_ref):
  p = p_ref[...].astype(jnp.float32)
  a = p[0] + p[1]
  h2 = jnp.maximum(s2_ref[...] + a * dinv_ref[...][:, 0:1], 0.0)
  logits = jnp.dot(h2, wo_ref[...], preferred_element_type=jnp.float32) + bo_ref[...]
  out_ref[...] = jnp.clip(logits, -4.0, 4.0)


_GRID = (N // BN,)
_FULL = lambda i: (0, 0)
_ROWB = lambda i: (i, 0)

_tc_pre = pl.pallas_call(
    _tc_pre_body,
    grid=_GRID,
    in_specs=[
        pl.BlockSpec((BN, D), _ROWB),
        pl.BlockSpec((D, H), _FULL),
    ],
    out_specs=pl.BlockSpec((BN, H), _ROWB),
    out_shape=jax.ShapeDtypeStruct((N, H), jnp.bfloat16),
)

_tc_mid = pl.pallas_call(
    _tc_mid_body,
    grid=_GRID,
    in_specs=[
        pl.BlockSpec((NC, BN, H), lambda i: (0, i, 0)),
        pl.BlockSpec((NC, BN, DW), lambda i: (0, i, 0)),
        pl.BlockSpec((BN, D), _ROWB),
        pl.BlockSpec((D, H), _FULL),
        pl.BlockSpec((1, H), _FULL),
        pl.BlockSpec((H, H), _FULL),
        pl.BlockSpec((H, H), _FULL),
        pl.BlockSpec((1, H), _FULL),
    ],
    out_specs=[pl.BlockSpec((BN, H), _ROWB), pl.BlockSpec((BN, H), _ROWB),
               pl.BlockSpec((BN, 8), _ROWB)],
    out_shape=[jax.ShapeDtypeStruct((N, H), jnp.bfloat16),
               jax.ShapeDtypeStruct((N, H), jnp.float32),
               jax.ShapeDtypeStruct((N, 8), jnp.float32)],
)

_tc_out = pl.pallas_call(
    _tc_out_body,
    grid=_GRID,
    in_specs=[
        pl.BlockSpec((NC, BN, H), lambda i: (0, i, 0)),
        pl.BlockSpec((BN, H), _ROWB),
        pl.BlockSpec((BN, 8), _ROWB),
        pl.BlockSpec((H, C), _FULL),
        pl.BlockSpec((1, C), _FULL),
    ],
    out_specs=pl.BlockSpec((BN, C), _ROWB),
    out_shape=jax.ShapeDtypeStruct((N, C), jnp.float32),
)


def kernel(x, edge_index, y, W1_self, W1_neigh, b1, W2_self, W2_neigh, b2,
           W_out, b_out):
  ei = edge_index.reshape(2, NC, NS, CHUNKS, CHUNK)
  zeros_h = jnp.zeros((ROWS, H), jnp.bfloat16)
  dzeros = jnp.zeros((ROWS, DW), jnp.float32)

  m1 = _tc_pre(x, W1_neigh)
  parts1, degp = _segsum_deg(m1, ei, zeros_h, dzeros)
  m2, s2, dinv = _tc_mid(parts1, degp, x, W1_self, b1.reshape(1, H),
                         W2_neigh, W2_self, b2.reshape(1, H))
  (parts2,) = _segsum_h(m2, ei, zeros_h)
  logits = _tc_out(parts2, s2, dinv, W_out, b_out.reshape(1, C))
  return (logits, y)
